# Initial kernel scaffold; baseline (speedup 1.0000x reference)
import jax
import jax.numpy as jnp
from jax.experimental import pallas as pl


def kernel(x, gate_W, fc1_w, fc1_b, fc2_w, fc2_b):
    raise NotImplementedError("write your pallas kernel here")


# speedup vs baseline: 2.2495x; 2.2495x over previous
import functools
import math

import jax
import jax.numpy as jnp
from jax.experimental import pallas as pl
from jax.experimental.pallas import tpu as pltpu

B, S, D = 1, 2048, 768
E, K, M = 8, 2, 3072
N = B * S
SIGMA = 1.0 / E
_INV_SQRT2 = 1.0 / math.sqrt(2.0)

TOK_TILE = 512
NT = N // TOK_TILE


def _router_body(x_ref, gw_ref, noise_ref, gating_ref, load_ref, wmat_ref):
    x = x_ref[...]
    gw = gw_ref[...]
    logits = jax.lax.dot_general(
        x, gw, (((1,), (0,)), ((), ())),
        preferred_element_type=jnp.float32,
    )
    m = jnp.max(logits, axis=-1, keepdims=True)
    ex = jnp.exp(logits - m)
    gating = ex / jnp.sum(ex, axis=-1, keepdims=True)
    gating_ref[...] = gating

    lane = jax.lax.broadcasted_iota(jnp.int32, (N, E), 1)
    g1 = jnp.max(gating, axis=-1, keepdims=True)
    i1 = jnp.min(jnp.where(gating == g1, lane, E), axis=-1, keepdims=True)
    masked = jnp.where(lane == i1, -jnp.inf, gating)
    g2 = jnp.max(masked, axis=-1, keepdims=True)
    i2 = jnp.min(jnp.where(masked == g2, lane, E), axis=-1, keepdims=True)
    denom = g1 + g2 + 1e-9
    w1 = g1 / denom
    w2 = g2 / denom
    wmat_ref[...] = jnp.where(lane == i1, w1, 0.0) + jnp.where(lane == i2, w2, 0.0)

    noisy = logits + noise_ref[...]
    n1 = jnp.max(noisy, axis=-1, keepdims=True)
    j1 = jnp.min(jnp.where(noisy == n1, lane, E), axis=-1, keepdims=True)
    nmasked = jnp.where(lane == j1, -jnp.inf, noisy)
    tau = jnp.max(nmasked, axis=-1, keepdims=True)
    z = (tau - logits) / SIGMA
    load_ref[...] = 1.0 - 0.5 * (1.0 + jax.lax.erf(z * _INV_SQRT2))


def _router(x_flat, gate_W, noise):
    return pl.pallas_call(
        _router_body,
        out_shape=(
            jax.ShapeDtypeStruct((N, E), jnp.float32),
            jax.ShapeDtypeStruct((N, E), jnp.float32),
            jax.ShapeDtypeStruct((N, E), jnp.float32),
        ),
    )(x_flat, gate_W, noise)


def _gelu(v):
    return 0.5 * v * (1.0 + jax.lax.erf(v * _INV_SQRT2))


def _dense_body(xb_ref, w1_ref, b1_ref, w2_ref, b2_ref, wmat_ref, out_ref):
    e = pl.program_id(0)
    t = pl.program_id(1)
    xb = xb_ref[...]
    h = jax.lax.dot_general(
        xb, w1_ref[0], (((1,), (0,)), ((), ())),
        preferred_element_type=jnp.float32,
    )
    h = _gelu(h + b1_ref[0])
    y = jax.lax.dot_general(
        h.astype(jnp.bfloat16), w2_ref[0], (((1,), (0,)), ((), ())),
        preferred_element_type=jnp.float32,
    )
    y = y + b2_ref[0]
    wblk = wmat_ref[...]
    lane = jax.lax.broadcasted_iota(jnp.int32, (TOK_TILE, E), 1)
    w_col = jnp.sum(jnp.where(lane == e, wblk, 0.0), axis=-1, keepdims=True)
    contrib = w_col * y
    ts = pl.ds(t * TOK_TILE, TOK_TILE)

    @pl.when(e == 0)
    def _():
        out_ref[ts, :] = contrib

    @pl.when(e != 0)
    def _():
        out_ref[ts, :] = out_ref[ts, :] + contrib


def _dense_experts(xb, fc1_w, fc1_b, fc2_w, fc2_b, wmat):
    return pl.pallas_call(
        _dense_body,
        grid=(E, NT),
        in_specs=[
            pl.BlockSpec((TOK_TILE, D), lambda e, t: (t, 0)),
            pl.BlockSpec((1, D, M), lambda e, t: (e, 0, 0)),
            pl.BlockSpec((1, 1, M), lambda e, t: (e, 0, 0)),
            pl.BlockSpec((1, M, D), lambda e, t: (e, 0, 0)),
            pl.BlockSpec((1, 1, D), lambda e, t: (e, 0, 0)),
            pl.BlockSpec((TOK_TILE, E), lambda e, t: (t, 0)),
        ],
        out_specs=pl.BlockSpec((N, D), lambda e, t: (0, 0)),
        out_shape=jax.ShapeDtypeStruct((N, D), jnp.float32),
        compiler_params=pltpu.CompilerParams(
            dimension_semantics=("arbitrary", "arbitrary"),
        ),
    )(xb, fc1_w, fc1_b, fc2_w, fc2_b, wmat)


def kernel(x, gate_W, fc1_w, fc1_b, fc2_w, fc2_b):
    x_flat = x.reshape(N, D)
    noise = jax.random.normal(jax.random.key(12345), (N, E), jnp.float32) * SIGMA
    gating, load_probs, wmat = _router(x_flat, gate_W, noise)
    xb = x_flat.astype(jnp.bfloat16)
    out = _dense_experts(
        xb,
        fc1_w.astype(jnp.bfloat16),
        fc1_b.reshape(E, 1, M),
        fc2_w.astype(jnp.bfloat16),
        fc2_b.reshape(E, 1, D),
        wmat,
    )
    return out.reshape(B, S, D), gating, load_probs
